# baseline (speedup 1.0000x reference)
import jax
import jax.numpy as jnp
from jax.experimental import pallas as pl
from jax.experimental.pallas import tpu as pltpu

_ROWS = 32768
_D = 768
_E = 8
_BLOCK_ROWS = 4096


_SPLIT = 3
_DC = _D // _SPLIT


def _gate_body(xa_ref, xb_ref, xc_ref, wt_ref, o_ref):
    acc = jnp.dot(xa_ref[...], wt_ref[0], preferred_element_type=jnp.float32)
    acc += jnp.dot(xb_ref[...], wt_ref[1], preferred_element_type=jnp.float32)
    acc += jnp.dot(xc_ref[...], wt_ref[2], preferred_element_type=jnp.float32)
    o_ref[...] = acc


def kernel(x, W):
    wt = W.T.reshape(_SPLIT, _DC, _E)
    grid = (_ROWS // _BLOCK_ROWS,)
    return pl.pallas_call(
        _gate_body,
        grid=grid,
        in_specs=[
            pl.BlockSpec((_BLOCK_ROWS, _DC), lambda i: (i, 0)),
            pl.BlockSpec((_BLOCK_ROWS, _DC), lambda i: (i, 1)),
            pl.BlockSpec((_BLOCK_ROWS, _DC), lambda i: (i, 2)),
            pl.BlockSpec((_SPLIT, _DC, _E), lambda i: (0, 0, 0)),
        ],
        out_specs=pl.BlockSpec((_BLOCK_ROWS, _E), lambda i: (i, 0)),
        out_shape=jax.ShapeDtypeStruct((_ROWS, _E), jnp.float32),
        compiler_params=pltpu.CompilerParams(
            dimension_semantics=("parallel",),
        ),
    )(x, x, x, wt)
